# baseline (speedup 1.0000x reference)
import jax
import jax.numpy as jnp
from jax import lax
from jax.experimental import pallas as pl
from jax.experimental.pallas import tpu as pltpu
from jax.experimental.pallas import tpu_sc as plsc

_B = 1024
_N = 100000
_MARGIN = 1.0
_MN = -50.0
_MX = 50.0
_TAU = 0.1

_NC = 2
_NSUB = 16
_NW = _NC * _NSUB
_L = 16
_T = 128
_NT = _N // _T
_TAIL = _NT * _T
_GK = (_B // 8) // _NW
_K = 11
_KW = 22
_NCH = 70
_PAIRS = 34


def _insert(a1, a2, a3, x):
    b1 = jnp.maximum(a1, x)
    r1 = jnp.minimum(a1, x)
    b2 = jnp.maximum(a2, r1)
    r2 = jnp.minimum(a2, r1)
    b3 = jnp.maximum(a3, r2)
    return b1, b2, b3


def _splat(x):
    return jnp.full((_L,), x, jnp.float32)


def _sc_body(sim_hbm, tgt_hbm, out_hbm,
             s0, s1, t0, t1, sw, tw, res_v, sem0, sem1, semw):
    cid = lax.axis_index("c")
    sid = lax.axis_index("s")
    wid = cid * _NSUB + sid
    g0 = wid * _GK

    sbufs = (s0, s1)
    tbufs = (t0, t1)
    sems = (sem0, sem1)

    def _slc(g, tile0, ntiles):
        row = pl.multiple_of((g0 + g) * 8, 8)
        col = pl.multiple_of(tile0 * _T, _T)
        return (pl.ds(row, 8), pl.ds(col, ntiles * _T))

    def start11(g, c, par):
        idx = _slc(g, c * _K, _K)
        pltpu.async_copy(sim_hbm.at[idx], sbufs[par], sems[par])
        pltpu.async_copy(tgt_hbm.at[idx], tbufs[par], sems[par])

    def wait11(g, c, par):
        idx = _slc(g, c * _K, _K)
        pltpu.make_async_copy(sim_hbm.at[idx], sbufs[par], sems[par]).wait()
        pltpu.make_async_copy(tgt_hbm.at[idx], tbufs[par], sems[par]).wait()

    def startw(g):
        idx = _slc(g, 69 * _K, _KW)
        pltpu.async_copy(sim_hbm.at[idx], sw, semw)
        pltpu.async_copy(tgt_hbm.at[idx], tw, semw)

    def waitw(g):
        idx = _slc(g, 69 * _K, _KW)
        pltpu.make_async_copy(sim_hbm.at[idx], sw, semw).wait()
        pltpu.make_async_copy(tgt_hbm.at[idx], tw, semw).wait()

    def proc(sbuf, tbuf, ntiles, acc):
        acc = list(acc)
        for r8 in range(8):
            def tloop(t, sub, r8=r8):
                a1, a2, a3, pa = sub
                for h in range(8):
                    off = t * _T + h * _L
                    x = sbuf[r8, pl.ds(off, _L)]
                    tt = tbuf[r8, pl.ds(off, _L)]
                    eq = tt == 1
                    xm = jnp.where(eq, jnp.float32(_MN), x)
                    xp = jnp.where(eq, x, jnp.float32(_MX))
                    pa = jnp.minimum(pa, xp)
                    a1, a2, a3 = _insert(a1, a2, a3, xm)
                return (a1, a2, a3, pa)
            acc[r8 * 4:(r8 + 1) * 4] = lax.fori_loop(
                0, ntiles, tloop, tuple(acc[r8 * 4:(r8 + 1) * 4]))
        return tuple(acc)

    start11(0, 0, 0)

    def group_body(gi, carry):
        acc = tuple([_splat(_MN), _splat(_MN), _splat(_MN), _splat(_MX)] * 8)

        def pair_body(j, acc):
            c = 2 * j
            wait11(gi, c, 0)
            start11(gi, c + 1, 1)
            acc = proc(s0, t0, _K, acc)
            wait11(gi, c + 1, 1)
            start11(gi, c + 2, 0)
            acc = proc(s1, t1, _K, acc)
            return acc

        acc = lax.fori_loop(0, _PAIRS, pair_body, acc)
        startw(gi)
        wait11(gi, 68, 0)
        acc = proc(s0, t0, _K, acc)

        @pl.when(gi < _GK - 1)
        def _():
            start11(gi + 1, 0, 0)

        waitw(gi)
        acc = proc(sw, tw, _KW, acc)

        for r8 in range(8):
            a1, a2, a3, pa = acc[r8 * 4:(r8 + 1) * 4]
            res_v[pl.ds(r8 * 4 * _L, _L)] = a1
            res_v[pl.ds(r8 * 4 * _L + _L, _L)] = a2
            res_v[pl.ds(r8 * 4 * _L + 2 * _L, _L)] = a3
            res_v[pl.ds(r8 * 4 * _L + 3 * _L, _L)] = pa
        off = pl.multiple_of((g0 + gi) * 8 * 4 * _L, 8)
        pltpu.sync_copy(res_v, out_hbm.at[pl.ds(off, 8 * 4 * _L)])
        return carry

    lax.fori_loop(0, _GK, group_body, jnp.int32(0))


def _tc_finalize(s_ref, st_ref, tt_ref, o_ref):
    s = s_ref[...]
    stail = st_ref[...]
    ttail = tt_ref[...]
    lane = lax.broadcasted_iota(jnp.int32, stail.shape, 1)
    valid = lane < (_N - _TAIL)
    eqt = jnp.logical_and(ttail == 1, valid)
    tail_top = jnp.where(valid, jnp.where(eqt, jnp.float32(_MN), stail),
                         jnp.float32(-1e30))
    tail_pos = jnp.where(eqt, stail, jnp.float32(_MX))
    top = jnp.concatenate([s[:, :3 * _L], tail_top], axis=1)
    posm = jnp.concatenate([s[:, 3 * _L:], tail_pos], axis=1)

    neg = jnp.float32(-1e30)
    p = jnp.min(posm, axis=1, keepdims=True)
    m1 = jnp.max(top, axis=1, keepdims=True)
    c1 = jnp.sum(jnp.where(top == m1, 1.0, 0.0), axis=1, keepdims=True)
    w2 = jnp.where(top < m1, top, neg)
    m2 = jnp.max(w2, axis=1, keepdims=True)
    c2 = jnp.sum(jnp.where(top == m2, 1.0, 0.0), axis=1, keepdims=True)
    w3 = jnp.where(top < m2, top, neg)
    m3 = jnp.max(w3, axis=1, keepdims=True)
    v1 = m1
    v2 = jnp.where(c1 >= 2.0, m1, m2)
    v3 = jnp.where(c1 >= 3.0, m1,
                   jnp.where(jnp.logical_or(c1 == 2.0, c2 >= 2.0), m2, m3))
    itau = jnp.float32(1.0 / _TAU)
    e1 = jnp.exp((v1 - m1) * itau)
    e2 = jnp.exp((v2 - m1) * itau)
    e3 = jnp.exp((v3 - m1) * itau)
    mg = jnp.float32(_MARGIN)
    l1 = jnp.maximum(v1 - p + mg, 0.0)
    l2 = jnp.maximum(v2 - p + mg, 0.0)
    l3 = jnp.maximum(v3 - p + mg, 0.0)
    row_loss = (l1 * e1 + l2 * e2 + l3 * e3) / (e1 + e2 + e3)
    o_ref[...] = (jnp.sum(row_loss) * jnp.float32(1.0 / (_B * 3.0)))[None, None]


@jax.jit
def kernel(sim_b, target):
    mesh = plsc.VectorSubcoreMesh(
        core_axis_name="c", subcore_axis_name="s",
        num_cores=_NC, num_subcores=_NSUB)
    survivors = pl.kernel(
        _sc_body,
        out_type=jax.ShapeDtypeStruct((_B * 4 * _L,), jnp.float32),
        mesh=mesh,
        scratch_types=[
            pltpu.VMEM((8, _K * _T), jnp.float32),
            pltpu.VMEM((8, _K * _T), jnp.float32),
            pltpu.VMEM((8, _K * _T), jnp.int32),
            pltpu.VMEM((8, _K * _T), jnp.int32),
            pltpu.VMEM((8, _KW * _T), jnp.float32),
            pltpu.VMEM((8, _KW * _T), jnp.int32),
            pltpu.VMEM((8 * 4 * _L,), jnp.float32),
            pltpu.SemaphoreType.DMA,
            pltpu.SemaphoreType.DMA,
            pltpu.SemaphoreType.DMA,
        ],
    )(sim_b, target)
    total = pl.pallas_call(
        _tc_finalize,
        grid=(1,),
        in_specs=[
            pl.BlockSpec((_B, 4 * _L), lambda i: (0, 0)),
            pl.BlockSpec((_B, _T), lambda i: (0, _NT)),
            pl.BlockSpec((_B, _T), lambda i: (0, _NT)),
        ],
        out_specs=pl.BlockSpec((1, 1), lambda i: (0, 0)),
        out_shape=jax.ShapeDtypeStruct((1, 1), jnp.float32),
    )(survivors.reshape(_B, 4 * _L), sim_b, target)
    return total[0, 0]


# speedup vs baseline: 4.2657x; 4.2657x over previous
import jax
import jax.numpy as jnp
from jax import lax
from jax.experimental import pallas as pl
from jax.experimental.pallas import tpu as pltpu
from jax.experimental.pallas import tpu_sc as plsc

_B = 1024
_N = 100000
_MARGIN = 1.0
_MN = -50.0
_MX = 50.0
_TAU = 0.1

_NC = 2
_NSUB = 16
_NW = _NC * _NSUB
_L = 16
_CS = 44000
_CB = 800
_TCN = (_N - _CS) // _CB
_NQ = 4
_CQ = _CS // _NQ
_RB = _B // 8
_CC = 200
_NCHK = _CQ // _CC
_PAIRS = (_NCHK - 1) // 2
_UC = 8
_NSETS = 4


def _insert(a1, a2, a3, x):
    b1 = jnp.maximum(a1, x)
    r1 = jnp.minimum(a1, x)
    b2 = jnp.maximum(a2, r1)
    r2 = jnp.minimum(a2, r1)
    b3 = jnp.maximum(a3, r2)
    return b1, b2, b3


def _splat(x):
    return jnp.full((_L,), x, jnp.float32)


def _sc_body(sim_hbm, tgt_hbm, out_hbm, s0, s1, t0, t1, res_v, sem0, sem1):
    cid = lax.axis_index("c")
    sid = lax.axis_index("s")
    wid = cid * _NSUB + sid
    q = wid // 8
    rt = wid % 8

    sbufs = (s0, s1)
    tbufs = (t0, t1)
    sems = (sem0, sem1)

    def _slc(c):
        col = pl.multiple_of(q * _CQ + c * _CC, 8)
        row = pl.multiple_of(rt * _RB, _RB)
        return (pl.ds(col, _CC), pl.ds(row, _RB))

    def start(c, par):
        idx = _slc(c)
        pltpu.async_copy(sim_hbm.at[idx], sbufs[par], sems[par])
        pltpu.async_copy(tgt_hbm.at[idx], tbufs[par], sems[par])

    def wait(c, par):
        idx = _slc(c)
        pltpu.make_async_copy(sim_hbm.at[idx], sbufs[par], sems[par]).wait()
        pltpu.make_async_copy(tgt_hbm.at[idx], tbufs[par], sems[par]).wait()

    def proc(sbuf, tbuf, acc):
        acc = list(acc)
        for s in range(8):
            def cloop(j, sub, s=s):
                sub = list(sub)
                c0 = j * _UC
                for u in range(_UC):
                    k = u % _NSETS
                    a1, a2, a3, pa = sub[k * 4:(k + 1) * 4]
                    x = sbuf[c0 + u, pl.ds(s * _L, _L)]
                    tt = tbuf[c0 + u, pl.ds(s * _L, _L)]
                    eq = tt == 1
                    xm = jnp.where(eq, jnp.float32(_MN), x)
                    xp = jnp.where(eq, x, jnp.float32(_MX))
                    pa = jnp.minimum(pa, xp)
                    a1, a2, a3 = _insert(a1, a2, a3, xm)
                    sub[k * 4:(k + 1) * 4] = [a1, a2, a3, pa]
                return tuple(sub)
            acc[s * 4 * _NSETS:(s + 1) * 4 * _NSETS] = lax.fori_loop(
                0, _CC // _UC, cloop,
                tuple(acc[s * 4 * _NSETS:(s + 1) * 4 * _NSETS]))
        return tuple(acc)

    start(0, 0)
    acc = tuple([_splat(_MN), _splat(_MN), _splat(_MN), _splat(_MX)]
                * (8 * _NSETS))

    def pair_body(j, acc):
        c = 2 * j
        wait(c, 0)
        start(c + 1, 1)
        acc = proc(s0, t0, acc)
        wait(c + 1, 1)
        start(c + 2, 0)
        acc = proc(s1, t1, acc)
        return acc

    acc = lax.fori_loop(0, _PAIRS, pair_body, acc)
    wait(_NCHK - 1, 0)
    acc = proc(s0, t0, acc)

    for s in range(8):
        sub = acc[s * 4 * _NSETS:(s + 1) * 4 * _NSETS]
        a1, a2, a3, pa = sub[0:4]
        for k in range(1, _NSETS):
            b1, b2, b3, pb = sub[k * 4:(k + 1) * 4]
            a1, a2, a3 = _insert(a1, a2, a3, b1)
            a1, a2, a3 = _insert(a1, a2, a3, b2)
            a1, a2, a3 = _insert(a1, a2, a3, b3)
            pa = jnp.minimum(pa, pb)
        res_v[pl.ds(s * _L, _L)] = a1
        res_v[pl.ds(_RB + s * _L, _L)] = a2
        res_v[pl.ds(2 * _RB + s * _L, _L)] = a3
        res_v[pl.ds(3 * _RB + s * _L, _L)] = pa
    for k in range(4):
        off = pl.multiple_of(q * 4 * _B + k * _B + rt * _RB, 8)
        pltpu.sync_copy(res_v.at[pl.ds(k * _RB, _RB)],
                        out_hbm.at[pl.ds(off, _RB)])


def _tc_scan(s_ref, t_ref, o_ref):
    i = pl.program_id(0)
    x = s_ref[...]
    t = t_ref[...]
    neg = jnp.float32(-1e30)
    eq = t == 1
    xm = jnp.where(eq, jnp.float32(_MN), x)
    xp = jnp.where(eq, x, jnp.float32(_MX))
    pb = jnp.min(xp, axis=0, keepdims=True)
    m1 = jnp.max(xm, axis=0, keepdims=True)
    c1 = jnp.sum(jnp.where(xm == m1, 1.0, 0.0), axis=0, keepdims=True)
    w2 = jnp.where(xm < m1, xm, neg)
    m2 = jnp.max(w2, axis=0, keepdims=True)
    c2 = jnp.sum(jnp.where(xm == m2, 1.0, 0.0), axis=0, keepdims=True)
    w3 = jnp.where(xm < m2, xm, neg)
    m3 = jnp.max(w3, axis=0, keepdims=True)
    v1 = m1
    v2 = jnp.where(c1 >= 2.0, m1, m2)
    v3 = jnp.where(c1 >= 3.0, m1,
                   jnp.where(jnp.logical_or(c1 == 2.0, c2 >= 2.0), m2, m3))

    @pl.when(i == 0)
    def _():
        o_ref[...] = jnp.concatenate([v1, v2, v3, pb], axis=0)

    @pl.when(i > 0)
    def _():
        a1 = o_ref[0:1, :]
        a2 = o_ref[1:2, :]
        a3 = o_ref[2:3, :]
        a1, a2, a3 = _insert(a1, a2, a3, v1)
        a1, a2, a3 = _insert(a1, a2, a3, v2)
        a1, a2, a3 = _insert(a1, a2, a3, v3)
        p = jnp.minimum(o_ref[3:4, :], pb)
        o_ref[...] = jnp.concatenate([a1, a2, a3, p], axis=0)


def _tc_finalize(x_ref, y_ref, o_ref):
    x = x_ref[...]
    y = y_ref[...]
    s4 = x.reshape(_NQ, 4, _B)
    tops = jnp.concatenate(
        [s4[:, :3, :].reshape(3 * _NQ, _B), y[0:3, :]], axis=0)
    pos = jnp.concatenate([s4[:, 3, :], y[3:4, :]], axis=0)
    neg = jnp.float32(-1e30)
    p = jnp.min(pos, axis=0)
    m1 = jnp.max(tops, axis=0)
    c1 = jnp.sum(jnp.where(tops == m1, 1.0, 0.0), axis=0)
    w2 = jnp.where(tops < m1, tops, neg)
    m2 = jnp.max(w2, axis=0)
    c2 = jnp.sum(jnp.where(tops == m2, 1.0, 0.0), axis=0)
    w3 = jnp.where(tops < m2, tops, neg)
    m3 = jnp.max(w3, axis=0)
    v1 = m1
    v2 = jnp.where(c1 >= 2.0, m1, m2)
    v3 = jnp.where(c1 >= 3.0, m1,
                   jnp.where(jnp.logical_or(c1 == 2.0, c2 >= 2.0), m2, m3))
    itau = jnp.float32(1.0 / _TAU)
    e1 = jnp.exp((v1 - m1) * itau)
    e2 = jnp.exp((v2 - m1) * itau)
    e3 = jnp.exp((v3 - m1) * itau)
    mg = jnp.float32(_MARGIN)
    l1 = jnp.maximum(v1 - p + mg, 0.0)
    l2 = jnp.maximum(v2 - p + mg, 0.0)
    l3 = jnp.maximum(v3 - p + mg, 0.0)
    row_loss = (l1 * e1 + l2 * e2 + l3 * e3) / (e1 + e2 + e3)
    o_ref[...] = (jnp.sum(row_loss) * jnp.float32(1.0 / (_B * 3.0)))[None, None]


@jax.jit
def kernel(sim_b, target):
    mesh = plsc.VectorSubcoreMesh(
        core_axis_name="c", subcore_axis_name="s",
        num_cores=_NC, num_subcores=_NSUB)
    partials = pl.kernel(
        _sc_body,
        out_type=jax.ShapeDtypeStruct((_NQ * 4 * _B,), jnp.float32),
        mesh=mesh,
        scratch_types=[
            pltpu.VMEM((_CC, _RB), jnp.float32),
            pltpu.VMEM((_CC, _RB), jnp.float32),
            pltpu.VMEM((_CC, _RB), jnp.int32),
            pltpu.VMEM((_CC, _RB), jnp.int32),
            pltpu.VMEM((4 * _RB,), jnp.float32),
            pltpu.SemaphoreType.DMA,
            pltpu.SemaphoreType.DMA,
        ],
    )(sim_b.T, target.T)
    tc_part = pl.pallas_call(
        _tc_scan,
        grid=(_TCN,),
        in_specs=[
            pl.BlockSpec((_CB, _B), lambda i: (_CS // _CB + i, 0)),
            pl.BlockSpec((_CB, _B), lambda i: (_CS // _CB + i, 0)),
        ],
        out_specs=pl.BlockSpec((4, _B), lambda i: (0, 0)),
        out_shape=jax.ShapeDtypeStruct((4, _B), jnp.float32),
    )(sim_b.T, target.T)
    total = pl.pallas_call(
        _tc_finalize,
        out_shape=jax.ShapeDtypeStruct((1, 1), jnp.float32),
    )(partials.reshape(4 * _NQ, _B), tc_part)
    return total[0, 0]
